# speedup vs baseline: 1.5205x; 1.5205x over previous
import functools

import jax
import jax.numpy as jnp
from jax import lax
from jax.experimental import pallas as pl
from jax.experimental.pallas import tpu as pltpu
from jax.experimental.pallas import tpu_sc as plsc


def _build_gather(n_rows, d):
    info = plsc.get_sparse_core_info()
    nc, ns = info.num_cores, info.num_subcores
    nw = nc * ns
    rows_per_w = n_rows // nw
    chunk = 32
    n_chunks = rows_per_w // chunk

    mesh = plsc.VectorSubcoreMesh(core_axis_name="c", subcore_axis_name="s")

    @functools.partial(
        pl.kernel,
        mesh=mesh,
        out_type=jax.ShapeDtypeStruct((n_rows, d), jnp.float32),
        scratch_types=[
            pltpu.VMEM((n_chunks, chunk), jnp.int32),
            pltpu.VMEM((chunk, d), jnp.float32),
            pltpu.VMEM((chunk, d), jnp.float32),
            pltpu.SemaphoreType.DMA,
            pltpu.SemaphoreType.DMA,
            pltpu.SemaphoreType.DMA,
            pltpu.SemaphoreType.DMA,
        ],
    )
    def gather_kernel(idx_hbm, table_hbm, out_hbm,
                      idx_v, buf0, buf1, gs0, gs1, os0, os1):
        wid = lax.axis_index("s") * nc + lax.axis_index("c")
        base = wid * rows_per_w
        pltpu.sync_copy(idx_hbm.at[wid], idx_v)

        bufs = (buf0, buf1)
        gsems = (gs0, gs1)
        osems = (os0, os1)
        gather = [None] * n_chunks
        out = [None] * n_chunks
        gather[0] = pltpu.async_copy(table_hbm.at[idx_v.at[0]], bufs[0], gsems[0])
        for c in range(n_chunks):
            cur = c % 2
            nxt = (c + 1) % 2
            if c + 1 < n_chunks:
                if c >= 1:
                    out[c - 1].wait()
                gather[c + 1] = pltpu.async_copy(
                    table_hbm.at[idx_v.at[c + 1]], bufs[nxt], gsems[nxt])
            gather[c].wait()
            out[c] = pltpu.async_copy(
                bufs[cur], out_hbm.at[pl.ds(base + c * chunk, chunk)], osems[cur])
        out[n_chunks - 2].wait()
        out[n_chunks - 1].wait()

    return gather_kernel, nw, n_chunks, chunk


def kernel(token_ids, table):
    b, s = token_ids.shape
    d = table.shape[1]
    n = b * s
    gather_fn, nw, n_chunks, chunk = _build_gather(n, d)
    idx = token_ids.astype(jnp.int32).reshape(nw, n_chunks, chunk)
    flat = gather_fn(idx, table)
    return flat.reshape(b, s, d)


# speedup vs baseline: 1.5654x; 1.0295x over previous
import functools

import jax
import jax.numpy as jnp
from jax import lax
from jax.experimental import pallas as pl
from jax.experimental.pallas import tpu as pltpu
from jax.experimental.pallas import tpu_sc as plsc


def _build_gather(n_rows, d):
    info = plsc.get_sparse_core_info()
    nc, ns = info.num_cores, info.num_subcores
    nw = nc * ns
    rows_per_w = n_rows // nw
    chunk = 32
    n_chunks = rows_per_w // chunk

    mesh = plsc.VectorSubcoreMesh(core_axis_name="c", subcore_axis_name="s")

    nbuf = 3

    @functools.partial(
        pl.kernel,
        mesh=mesh,
        out_type=jax.ShapeDtypeStruct((n_rows, d), jnp.float32),
        scratch_types=(
            [pltpu.VMEM((n_chunks, chunk), jnp.int32)]
            + [pltpu.VMEM((chunk, d), jnp.float32) for _ in range(nbuf)]
            + [pltpu.SemaphoreType.DMA for _ in range(2 * nbuf)]
        ),
    )
    def gather_kernel(idx_hbm, table_hbm, out_hbm, idx_v, *scratch):
        bufs = scratch[:nbuf]
        gsems = scratch[nbuf:2 * nbuf]
        osems = scratch[2 * nbuf:]
        wid = lax.axis_index("s") * nc + lax.axis_index("c")
        base = wid * rows_per_w
        pltpu.sync_copy(idx_hbm.at[wid], idx_v)

        gather = [None] * n_chunks
        out = [None] * n_chunks
        for c in range(min(nbuf, n_chunks)):
            gather[c] = pltpu.async_copy(
                table_hbm.at[idx_v.at[c]], bufs[c % nbuf], gsems[c % nbuf])
        for c in range(n_chunks):
            if c >= 1 and (c - 1) + nbuf < n_chunks:
                out[c - 1].wait()
                g = (c - 1) + nbuf
                gather[g] = pltpu.async_copy(
                    table_hbm.at[idx_v.at[g]], bufs[g % nbuf], gsems[g % nbuf])
            gather[c].wait()
            out[c] = pltpu.async_copy(
                bufs[c % nbuf], out_hbm.at[pl.ds(base + c * chunk, chunk)],
                osems[c % nbuf])
        for c in range(max(0, n_chunks - nbuf), n_chunks):
            out[c].wait()

    return gather_kernel, nw, n_chunks, chunk


def kernel(token_ids, table):
    b, s = token_ids.shape
    d = table.shape[1]
    n = b * s
    gather_fn, nw, n_chunks, chunk = _build_gather(n, d)
    idx = token_ids.astype(jnp.int32).reshape(nw, n_chunks, chunk)
    flat = gather_fn(idx, table)
    return flat.reshape(b, s, d)


# speedup vs baseline: 1.5687x; 1.0021x over previous
import functools

import jax
import jax.numpy as jnp
from jax import lax
from jax.experimental import pallas as pl
from jax.experimental.pallas import tpu as pltpu
from jax.experimental.pallas import tpu_sc as plsc


def _build_gather(n_rows, d):
    info = plsc.get_sparse_core_info()
    nc, ns = info.num_cores, info.num_subcores
    nw = nc * ns
    rows_per_w = n_rows // nw
    chunks = [56, 56, 56, 56, 32]
    assert sum(chunks) == rows_per_w
    max_chunk = max(chunks)
    n_chunks = len(chunks)
    offs = [sum(chunks[:i]) for i in range(n_chunks)]

    mesh = plsc.VectorSubcoreMesh(core_axis_name="c", subcore_axis_name="s")

    nbuf = 2

    @functools.partial(
        pl.kernel,
        mesh=mesh,
        out_type=jax.ShapeDtypeStruct((n_rows, d), jnp.float32),
        scratch_types=(
            [pltpu.VMEM((rows_per_w,), jnp.int32)]
            + [pltpu.VMEM((max_chunk, d), jnp.float32) for _ in range(nbuf)]
            + [pltpu.SemaphoreType.DMA for _ in range(2 * nbuf)]
        ),
    )
    def gather_kernel(idx_hbm, table_hbm, out_hbm, idx_v, *scratch):
        bufs = scratch[:nbuf]
        gsems = scratch[nbuf:2 * nbuf]
        osems = scratch[2 * nbuf:]
        wid = lax.axis_index("s") * nc + lax.axis_index("c")
        base = wid * rows_per_w
        pltpu.sync_copy(idx_hbm.at[wid], idx_v)

        def start_gather(c):
            b = c % nbuf
            return pltpu.async_copy(
                table_hbm.at[idx_v.at[pl.ds(offs[c], chunks[c])]],
                bufs[b].at[pl.ds(0, chunks[c])], gsems[b])

        gather = [None] * n_chunks
        out = [None] * n_chunks
        for c in range(min(nbuf, n_chunks)):
            gather[c] = start_gather(c)
        for c in range(n_chunks):
            if c >= 1 and (c - 1) + nbuf < n_chunks:
                out[c - 1].wait()
                gather[(c - 1) + nbuf] = start_gather((c - 1) + nbuf)
            gather[c].wait()
            out[c] = pltpu.async_copy(
                bufs[c % nbuf].at[pl.ds(0, chunks[c])],
                out_hbm.at[pl.ds(base + offs[c], chunks[c])],
                osems[c % nbuf])
        for c in range(max(0, n_chunks - nbuf), n_chunks):
            out[c].wait()

    return gather_kernel, nw


def kernel(token_ids, table):
    b, s = token_ids.shape
    d = table.shape[1]
    n = b * s
    gather_fn, nw = _build_gather(n, d)
    idx = token_ids.astype(jnp.int32).reshape(nw, n // nw)
    flat = gather_fn(idx, table)
    return flat.reshape(b, s, d)
